# Initial kernel scaffold; baseline (speedup 1.0000x reference)
import jax
import jax.numpy as jnp
from jax.experimental import pallas as pl


def kernel(xyz, features, W0, gamma0, beta0, W1, gamma1, beta1):
    raise NotImplementedError("write your pallas kernel here")


# speedup vs baseline: 30.6819x; 30.6819x over previous
import functools

import jax
import jax.numpy as jnp
from jax import lax
from jax.experimental import pallas as pl
from jax.experimental.pallas import tpu as pltpu
from jax.experimental.pallas import tpu_sc as plsc

_B = 2
_N = 16384
_S = 1024
_R2 = (0.2 * 0.2, 0.4 * 0.4)
_K = (16, 32)
_NW = 32
_SUB = 8
_LN = 2048


def _proj_body(xyz_ref, feat_ref, w0x_ref, w0f_ref, w1x_ref, w1f_ref,
               p0_ref, p1_ref):
    x = xyz_ref[0]
    f = feat_ref[0]
    p0_ref[0] = (jnp.dot(f, w0f_ref[...], preferred_element_type=jnp.float32)
                 + jnp.dot(x, w0x_ref[...], preferred_element_type=jnp.float32))
    p1_ref[0] = (jnp.dot(f, w1f_ref[...], preferred_element_type=jnp.float32)
                 + jnp.dot(x, w1x_ref[...], preferred_element_type=jnp.float32))


def _run_proj(xyz, features, W0, W1):
    w0x, w0f = W0[:3], W0[3:]
    w1x, w1f = W1[:3], W1[3:]
    nblk = _N // 1024
    grid = (_B, nblk)
    wspec3 = pl.BlockSpec((3, 64), lambda b, n: (0, 0))
    wspec128 = pl.BlockSpec((128, 64), lambda b, n: (0, 0))
    return pl.pallas_call(
        _proj_body,
        grid=grid,
        in_specs=[
            pl.BlockSpec((1, 1024, 3), lambda b, n: (b, n, 0)),
            pl.BlockSpec((1, 1024, 128), lambda b, n: (b, n, 0)),
            wspec3, wspec128, wspec3, wspec128,
        ],
        out_specs=[
            pl.BlockSpec((1, 1024, 64), lambda b, n: (b, n, 0)),
            pl.BlockSpec((1, 1024, 64), lambda b, n: (b, n, 0)),
        ],
        out_shape=[
            jax.ShapeDtypeStruct((_B, _N, 64), jnp.float32),
            jax.ShapeDtypeStruct((_B, _N, 64), jnp.float32),
        ],
    )(xyz, features, w0x, w0f, w1x, w1f)


def _fps_body(xyzr_ref, newxyz_ref, dist_ref):
    gidx = (lax.broadcasted_iota(jnp.int32, (_SUB, _LN), 0) * _LN
            + lax.broadcasted_iota(jnp.int32, (_SUB, _LN), 1))
    dist_ref[...] = jnp.full((_B, _SUB, _LN), 1e10, jnp.float32)

    def step(i, fars):
        new_fars = []
        for b in range(_B):
            far = fars[b]
            xb = xyzr_ref[b, 0]
            yb = xyzr_ref[b, 1]
            zb = xyzr_ref[b, 2]
            sel = gidx == far
            cx = jnp.max(jnp.where(sel, xb, -1e30))
            cy = jnp.max(jnp.where(sel, yb, -1e30))
            cz = jnp.max(jnp.where(sel, zb, -1e30))
            row = jnp.concatenate(
                [jnp.full((1, 1), cx), jnp.full((1, 1), cy),
                 jnp.full((1, 1), cz)], axis=1)
            newxyz_ref[b, pl.ds(i, 1), :] = row
            d = (xb - cx) ** 2 + (yb - cy) ** 2 + (zb - cz) ** 2
            dmin = jnp.minimum(dist_ref[b], d)
            dist_ref[b] = dmin
            m = jnp.max(dmin)
            cand = jnp.where(dmin == m, gidx, jnp.int32(2 ** 30))
            new_fars.append(jnp.min(cand))
        return tuple(new_fars)

    lax.fori_loop(0, _S, step, tuple(jnp.int32(0) for _ in range(_B)))


def _run_fps(xyzr):
    return pl.pallas_call(
        _fps_body,
        out_shape=jax.ShapeDtypeStruct((_B, _S, 3), jnp.float32),
        scratch_shapes=[pltpu.VMEM((_B, _SUB, _LN), jnp.float32)],
    )(xyzr)


def _sc_mesh():
    return plsc.VectorSubcoreMesh(core_axis_name="c", subcore_axis_name="s",
                                  num_cores=2, num_subcores=16)


def _bq_body(xyzt, newt, out0, out1, xs, ys, zs, cxr, cyr, czr, cbuf, ob0, ob1):
    wid = lax.axis_index("s") * 2 + lax.axis_index("c")
    b = wid // 16
    part = wid % 16
    s0 = part * 64
    pltpu.sync_copy(xyzt.at[pl.ds((b * 3 + 0) * _N, _N)], xs)
    pltpu.sync_copy(xyzt.at[pl.ds((b * 3 + 1) * _N, _N)], ys)
    pltpu.sync_copy(xyzt.at[pl.ds((b * 3 + 2) * _N, _N)], zs)
    pltpu.sync_copy(newt.at[pl.ds((b * 3 + 0) * _S + s0, 64)], cxr)
    pltpu.sync_copy(newt.at[pl.ds((b * 3 + 1) * _S + s0, 64)], cyr)
    pltpu.sync_copy(newt.at[pl.ds((b * 3 + 2) * _S + s0, 64)], czr)
    lane = lax.iota(jnp.int32, 16)
    base = b * _N

    for scale in range(2):
        r2 = jnp.float32(_R2[scale])
        K = _K[scale]
        ob = (ob0, ob1)[scale]

        def center_body(i, _, K=K, r2=r2, ob=ob):
            idxv = jnp.full((16,), 0, jnp.int32) + i
            cxv = plsc.load_gather(cxr, [idxv])
            cyv = plsc.load_gather(cyr, [idxv])
            czv = plsc.load_gather(czr, [idxv])
            cbuf[pl.ds(0, 16)] = jnp.full((16,), base, jnp.int32)

            def cond(carry):
                j, cnt = carry
                return jnp.logical_and(cnt < K, j < _N // 16)

            def body(carry):
                j, cnt = carry
                xv = xs[pl.ds(j * 16, 16)]
                yv = ys[pl.ds(j * 16, 16)]
                zv = zs[pl.ds(j * 16, 16)]
                dx = xv - cxv
                dy = yv - cyv
                dz = zv - czv
                d = dx * dx + dy * dy + dz * dz
                msk = d < r2
                gi = lane + (j * 16 + base)
                plsc.store_compressed(cbuf.at[pl.ds(cnt, 16)], gi, mask=msk)
                cnt = cnt + jnp.sum(msk.astype(jnp.int32))
                return j + 1, cnt

            _, cnt = lax.while_loop(cond, body, (jnp.int32(0), jnp.int32(0)))
            firstv = plsc.load_gather(cbuf, [jnp.zeros((16,), jnp.int32)])
            v0 = cbuf[pl.ds(0, 16)]
            if K == 16:
                ob[pl.ds(i * 16, 16)] = jnp.where(lane < cnt, v0, firstv)
            else:
                v1 = cbuf[pl.ds(16, 16)]
                ob[pl.ds(i * 32, 16)] = jnp.where(lane < cnt, v0, firstv)
                ob[pl.ds(i * 32 + 16, 16)] = jnp.where(lane + 16 < cnt, v1,
                                                       firstv)
            return 0

        lax.fori_loop(0, 64, center_body, 0)
        out = (out0, out1)[scale]
        pltpu.sync_copy(ob, out.at[pl.ds(b * _S * K + part * 64 * K, 64 * K)])


def _run_ballquery(xyzt, newt):
    f = functools.partial(
        pl.kernel, _bq_body,
        out_type=(jax.ShapeDtypeStruct((_B * _S * 16,), jnp.int32),
                  jax.ShapeDtypeStruct((_B * _S * 32,), jnp.int32)),
        mesh=_sc_mesh(),
        compiler_params=pltpu.CompilerParams(needs_layout_passes=False),
        scratch_types=[
            pltpu.VMEM((_N,), jnp.float32),
            pltpu.VMEM((_N,), jnp.float32),
            pltpu.VMEM((_N,), jnp.float32),
            pltpu.VMEM((64,), jnp.float32),
            pltpu.VMEM((64,), jnp.float32),
            pltpu.VMEM((64,), jnp.float32),
            pltpu.VMEM((64,), jnp.int32),
            pltpu.VMEM((64 * 16,), jnp.int32),
            pltpu.VMEM((64 * 32,), jnp.int32),
        ],
    )
    return f()(xyzt, newt)


def _gr_compute(rows, stage_max, stage_min, stage_sum, ssq, K, ngroups, goff):
    def gbody(g, ssq_c):
        first = g * K
        mx = [rows[first, pl.ds(c * 16, 16)] for c in range(4)]
        mn = list(mx)
        sm = list(mx)
        sq = [ssq_c[c] + mx[c] * mx[c] for c in range(4)]
        for r in range(1, K):
            for c in range(4):
                v = rows[first + r, pl.ds(c * 16, 16)]
                mx[c] = jnp.maximum(mx[c], v)
                mn[c] = jnp.minimum(mn[c], v)
                sm[c] = sm[c] + v
                sq[c] = sq[c] + v * v
        for c in range(4):
            o = (goff + g) * 64 + c * 16
            stage_max[pl.ds(o, 16)] = mx[c]
            stage_min[pl.ds(o, 16)] = mn[c]
            stage_sum[pl.ds(o, 16)] = sm[c]
        return tuple(sq)

    return lax.fori_loop(0, ngroups, gbody, ssq)


def _gr_body(pj0, pj1, gi0, gi1, pmax0, pmin0, gsum0, ssq0,
             pmax1, pmin1, gsum1, ssq1,
             idxb, rows, smax, smin, ssum, sem):
    wid = lax.axis_index("s") * 2 + lax.axis_index("c")

    pltpu.sync_copy(gi0.at[pl.ds(wid * 8, 8)], idxb)
    descs = [pltpu.async_copy(pj0.at[idxb.at[j]],
                              rows.at[pl.ds(j * 128, 128)], sem)
             for j in range(8)]
    for dsc in descs:
        dsc.wait()
    zero = jnp.zeros((16,), jnp.float32)
    sq = _gr_compute(rows, smax, smin, ssum, (zero,) * 4, 16, 64, 0)
    for c in range(4):
        smax[pl.ds(64 * 64 + c * 16, 16)] = sq[c]
    pltpu.sync_copy(smax.at[pl.ds(0, 64 * 64)], pmax0.at[pl.ds(wid * 64 * 64, 64 * 64)])
    pltpu.sync_copy(smin.at[pl.ds(0, 64 * 64)], pmin0.at[pl.ds(wid * 64 * 64, 64 * 64)])
    pltpu.sync_copy(ssum.at[pl.ds(0, 64 * 64)], gsum0.at[pl.ds(wid * 64 * 64, 64 * 64)])
    pltpu.sync_copy(smax.at[pl.ds(64 * 64, 64)], ssq0.at[pl.ds(wid * 64, 64)])

    sq = (zero,) * 4
    for h in range(2):
        pltpu.sync_copy(gi1.at[pl.ds(wid * 16 + h * 8, 8)], idxb)
        descs = [pltpu.async_copy(pj1.at[idxb.at[j]],
                                  rows.at[pl.ds(j * 128, 128)], sem)
                 for j in range(8)]
        for dsc in descs:
            dsc.wait()
        sq = _gr_compute(rows, smax, smin, ssum, sq, 32, 32, h * 32)
    for c in range(4):
        smax[pl.ds(64 * 64 + c * 16, 16)] = sq[c]
    pltpu.sync_copy(smax.at[pl.ds(0, 64 * 64)], pmax1.at[pl.ds(wid * 64 * 64, 64 * 64)])
    pltpu.sync_copy(smin.at[pl.ds(0, 64 * 64)], pmin1.at[pl.ds(wid * 64 * 64, 64 * 64)])
    pltpu.sync_copy(ssum.at[pl.ds(0, 64 * 64)], gsum1.at[pl.ds(wid * 64 * 64, 64 * 64)])
    pltpu.sync_copy(smax.at[pl.ds(64 * 64, 64)], ssq1.at[pl.ds(wid * 64, 64)])


def _run_gatherreduce(pj0, pj1, gi0, gi1):
    flat = jax.ShapeDtypeStruct((_B * _S * 64,), jnp.float32)
    sqs = jax.ShapeDtypeStruct((_NW * 64,), jnp.float32)
    f = functools.partial(
        pl.kernel, _gr_body,
        out_type=(flat, flat, flat, sqs, flat, flat, flat, sqs),
        mesh=_sc_mesh(),
        compiler_params=pltpu.CompilerParams(needs_layout_passes=False,
                                             use_tc_tiling_on_sc=False),
        scratch_types=[
            pltpu.VMEM((8, 128), jnp.int32),
            pltpu.VMEM((1024, 64), jnp.float32),
            pltpu.VMEM((64 * 64 + 64,), jnp.float32),
            pltpu.VMEM((64 * 64,), jnp.float32),
            pltpu.VMEM((64 * 64,), jnp.float32),
            pltpu.SemaphoreType.DMA,
        ],
    )
    return f()(pj0, pj1, gi0, gi1)


def _fin_body(newt_ref, pmax0_ref, pmin0_ref, gsum0_ref, ssq0_ref,
              pmax1_ref, pmin1_ref, gsum1_ref, ssq1_ref,
              w0x_ref, w1x_ref, g0_ref, b0_ref, g1_ref, b1_ref, out_ref):
    dn = (((0,), (0,)), ((), ()))
    for scale in range(2):
        K = _K[scale]
        pmax = (pmax0_ref, pmax1_ref)[scale]
        pmin = (pmin0_ref, pmin1_ref)[scale]
        gsum = (gsum0_ref, gsum1_ref)[scale]
        ssq = (ssq0_ref, ssq1_ref)[scale]
        wx = (w0x_ref, w1x_ref)[scale][...]
        gam = (g0_ref, g1_ref)[scale][...]
        bet = (b0_ref, b1_ref)[scale][...]
        R = _B * _S * K
        corr = [lax.dot_general(newt_ref[b], wx, dn,
                                preferred_element_type=jnp.float32)
                for b in range(_B)]
        sum_corr = sum(jnp.sum(c, axis=0) for c in corr)
        sum_gsum = jnp.sum(gsum[...], axis=(0, 1))
        cross = sum(jnp.sum(corr[b] * gsum[b], axis=0) for b in range(_B))
        sum_cc = sum(jnp.sum(c * c, axis=0) for c in corr)
        sumsq = jnp.sum(ssq[...], axis=0)
        mean = (sum_gsum - K * sum_corr) / R
        esq = (sumsq - 2.0 * cross + K * sum_cc) / R
        var = esq - mean * mean
        inv = lax.rsqrt(var + 1e-5)
        for b in range(_B):
            z = jnp.where(gam >= 0.0, pmax[b] - corr[b], pmin[b] - corr[b])
            y = jnp.maximum((z - mean) * inv * gam + bet, 0.0)
            out_ref[b, :, scale * 64:(scale + 1) * 64] = y


def _run_finalize(newt, pmax0, pmin0, gsum0, ssq0, pmax1, pmin1, gsum1, ssq1,
                  W0, W1, gamma0, beta0, gamma1, beta1):
    return pl.pallas_call(
        _fin_body,
        out_shape=jax.ShapeDtypeStruct((_B, _S, 128), jnp.float32),
    )(newt, pmax0, pmin0, gsum0, ssq0, pmax1, pmin1, gsum1, ssq1,
      W0[:3], W1[:3], gamma0, beta0, gamma1, beta1)


def kernel(xyz, features, W0, gamma0, beta0, W1, gamma1, beta1):
    xyzt = jnp.transpose(xyz, (0, 2, 1))
    xyzr = xyzt.reshape(_B, 3, _SUB, _LN)

    pj0, pj1 = _run_proj(xyz, features, W0, W1)
    new_xyz = _run_fps(xyzr)
    newt = jnp.transpose(new_xyz, (0, 2, 1))

    gi0, gi1 = _run_ballquery(xyzt.reshape(-1), newt.reshape(-1))
    gi0 = gi0.reshape(_B * _S * 16 // 128, 128)
    gi1 = gi1.reshape(_B * _S * 32 // 128, 128)

    (pmax0, pmin0, gsum0, ssq0, pmax1, pmin1, gsum1, ssq1) = _run_gatherreduce(
        pj0.reshape(_B * _N, 64), pj1.reshape(_B * _N, 64), gi0, gi1)

    shp = (_B, _S, 64)
    out = _run_finalize(newt, pmax0.reshape(shp), pmin0.reshape(shp),
                        gsum0.reshape(shp), ssq0.reshape(_NW, 64),
                        pmax1.reshape(shp), pmin1.reshape(shp),
                        gsum1.reshape(shp), ssq1.reshape(_NW, 64),
                        W0, W1, gamma0, beta0, gamma1, beta1)
    new_features = jnp.transpose(out, (0, 2, 1))
    return new_xyz, new_features


# speedup vs baseline: 49.2300x; 1.6045x over previous
import functools

import jax
import jax.numpy as jnp
from jax import lax
from jax.experimental import pallas as pl
from jax.experimental.pallas import tpu as pltpu
from jax.experimental.pallas import tpu_sc as plsc

_B = 2
_N = 16384
_S = 1024
_R2 = (0.2 * 0.2, 0.4 * 0.4)
_K = (16, 32)
_NW = 32
_SUB = 8
_LN = 2048


def _proj_body(xyz_ref, feat_ref, w0x_ref, w0f_ref, w1x_ref, w1f_ref,
               p0_ref, p1_ref):
    x = xyz_ref[0]
    f = feat_ref[0]
    p0_ref[0] = (jnp.dot(f, w0f_ref[...], preferred_element_type=jnp.float32)
                 + jnp.dot(x, w0x_ref[...], preferred_element_type=jnp.float32))
    p1_ref[0] = (jnp.dot(f, w1f_ref[...], preferred_element_type=jnp.float32)
                 + jnp.dot(x, w1x_ref[...], preferred_element_type=jnp.float32))


def _run_proj(xyz, features, W0, W1):
    w0x, w0f = W0[:3], W0[3:]
    w1x, w1f = W1[:3], W1[3:]
    nblk = _N // 1024
    grid = (_B, nblk)
    wspec3 = pl.BlockSpec((3, 64), lambda b, n: (0, 0))
    wspec128 = pl.BlockSpec((128, 64), lambda b, n: (0, 0))
    return pl.pallas_call(
        _proj_body,
        grid=grid,
        in_specs=[
            pl.BlockSpec((1, 1024, 3), lambda b, n: (b, n, 0)),
            pl.BlockSpec((1, 1024, 128), lambda b, n: (b, n, 0)),
            wspec3, wspec128, wspec3, wspec128,
        ],
        out_specs=[
            pl.BlockSpec((1, 1024, 64), lambda b, n: (b, n, 0)),
            pl.BlockSpec((1, 1024, 64), lambda b, n: (b, n, 0)),
        ],
        out_shape=[
            jax.ShapeDtypeStruct((_B, _N, 64), jnp.float32),
            jax.ShapeDtypeStruct((_B, _N, 64), jnp.float32),
        ],
    )(xyz, features, w0x, w0f, w1x, w1f)


def _fps_body(xyzr_ref, newxyz_ref, dist_ref):
    dist_ref[...] = jnp.full((_B, _SUB, _LN), 1e10, jnp.float32)

    def step(i, carry):
        nxt = []
        for b in range(_B):
            cx, cy, cz = carry[b]
            row = jnp.concatenate(
                [jnp.full((1, 1), cx), jnp.full((1, 1), cy),
                 jnp.full((1, 1), cz)], axis=1)
            newxyz_ref[b, pl.ds(i, 1), :] = row
            xb = xyzr_ref[b, 0]
            yb = xyzr_ref[b, 1]
            zb = xyzr_ref[b, 2]
            d = (xb - cx) ** 2 + (yb - cy) ** 2 + (zb - cz) ** 2
            dmin = jnp.minimum(dist_ref[b], d)
            dist_ref[b] = dmin
            m = jnp.max(dmin)
            sel = dmin == m
            nxt.append((jnp.max(jnp.where(sel, xb, -1e30)),
                        jnp.max(jnp.where(sel, yb, -1e30)),
                        jnp.max(jnp.where(sel, zb, -1e30))))
        return tuple(nxt)

    init = tuple((xyzr_ref[b, 0, 0, 0], xyzr_ref[b, 1, 0, 0],
                  xyzr_ref[b, 2, 0, 0]) for b in range(_B))
    lax.fori_loop(0, _S, step, init)


def _run_fps(xyzr):
    return pl.pallas_call(
        _fps_body,
        out_shape=jax.ShapeDtypeStruct((_B, _S, 3), jnp.float32),
        scratch_shapes=[pltpu.VMEM((_B, _SUB, _LN), jnp.float32)],
    )(xyzr)


def _sc_mesh():
    return plsc.VectorSubcoreMesh(core_axis_name="c", subcore_axis_name="s",
                                  num_cores=2, num_subcores=16)


def _bq_body(xyzt, newt, out0, out1, xs, ys, zs, cxr, cyr, czr, cbuf, ob0, ob1):
    wid = lax.axis_index("s") * 2 + lax.axis_index("c")
    b = wid // 16
    part = wid % 16
    s0 = part * 64
    pltpu.sync_copy(xyzt.at[pl.ds((b * 3 + 0) * _N, _N)], xs)
    pltpu.sync_copy(xyzt.at[pl.ds((b * 3 + 1) * _N, _N)], ys)
    pltpu.sync_copy(xyzt.at[pl.ds((b * 3 + 2) * _N, _N)], zs)
    pltpu.sync_copy(newt.at[pl.ds((b * 3 + 0) * _S + s0, 64)], cxr)
    pltpu.sync_copy(newt.at[pl.ds((b * 3 + 1) * _S + s0, 64)], cyr)
    pltpu.sync_copy(newt.at[pl.ds((b * 3 + 2) * _S + s0, 64)], czr)
    lane = lax.iota(jnp.int32, 16)
    base = b * _N

    for scale in range(2):
        r2 = jnp.float32(_R2[scale])
        K = _K[scale]
        ob = (ob0, ob1)[scale]

        def center_body(i, _, K=K, r2=r2, ob=ob):
            idxv = jnp.full((16,), 0, jnp.int32) + i
            cxv = plsc.load_gather(cxr, [idxv])
            cyv = plsc.load_gather(cyr, [idxv])
            czv = plsc.load_gather(czr, [idxv])
            cbuf[pl.ds(0, 16)] = jnp.full((16,), base, jnp.int32)

            def cond(carry):
                j, cnt = carry
                return jnp.logical_and(cnt < K, j < _N // 16)

            def body(carry):
                j, cnt = carry
                xv = xs[pl.ds(j * 16, 16)]
                yv = ys[pl.ds(j * 16, 16)]
                zv = zs[pl.ds(j * 16, 16)]
                dx = xv - cxv
                dy = yv - cyv
                dz = zv - czv
                d = dx * dx + dy * dy + dz * dz
                msk = d < r2
                gi = lane + (j * 16 + base)
                plsc.store_compressed(cbuf.at[pl.ds(cnt, 16)], gi, mask=msk)
                cnt = cnt + jnp.sum(msk.astype(jnp.int32))
                return j + 1, cnt

            _, cnt = lax.while_loop(cond, body, (jnp.int32(0), jnp.int32(0)))
            firstv = plsc.load_gather(cbuf, [jnp.zeros((16,), jnp.int32)])
            v0 = cbuf[pl.ds(0, 16)]
            if K == 16:
                ob[pl.ds(i * 16, 16)] = jnp.where(lane < cnt, v0, firstv)
            else:
                v1 = cbuf[pl.ds(16, 16)]
                ob[pl.ds(i * 32, 16)] = jnp.where(lane < cnt, v0, firstv)
                ob[pl.ds(i * 32 + 16, 16)] = jnp.where(lane + 16 < cnt, v1,
                                                       firstv)
            return 0

        lax.fori_loop(0, 64, center_body, 0)
        out = (out0, out1)[scale]
        pltpu.sync_copy(ob, out.at[pl.ds(b * _S * K + part * 64 * K, 64 * K)])


def _run_ballquery(xyzt, newt):
    f = functools.partial(
        pl.kernel, _bq_body,
        out_type=(jax.ShapeDtypeStruct((_B * _S * 16,), jnp.int32),
                  jax.ShapeDtypeStruct((_B * _S * 32,), jnp.int32)),
        mesh=_sc_mesh(),
        compiler_params=pltpu.CompilerParams(needs_layout_passes=False),
        scratch_types=[
            pltpu.VMEM((_N,), jnp.float32),
            pltpu.VMEM((_N,), jnp.float32),
            pltpu.VMEM((_N,), jnp.float32),
            pltpu.VMEM((64,), jnp.float32),
            pltpu.VMEM((64,), jnp.float32),
            pltpu.VMEM((64,), jnp.float32),
            pltpu.VMEM((64,), jnp.int32),
            pltpu.VMEM((64 * 16,), jnp.int32),
            pltpu.VMEM((64 * 32,), jnp.int32),
        ],
    )
    return f()(xyzt, newt)


def _gr_compute(rows, stage_max, stage_min, stage_sum, ssq, K, ngroups, goff):
    def gbody(g, ssq_c):
        first = g * K
        mx = [rows[first, pl.ds(c * 16, 16)] for c in range(4)]
        mn = list(mx)
        sm = list(mx)
        sq = [ssq_c[c] + mx[c] * mx[c] for c in range(4)]
        for r in range(1, K):
            for c in range(4):
                v = rows[first + r, pl.ds(c * 16, 16)]
                mx[c] = jnp.maximum(mx[c], v)
                mn[c] = jnp.minimum(mn[c], v)
                sm[c] = sm[c] + v
                sq[c] = sq[c] + v * v
        for c in range(4):
            o = (goff + g) * 64 + c * 16
            stage_max[pl.ds(o, 16)] = mx[c]
            stage_min[pl.ds(o, 16)] = mn[c]
            stage_sum[pl.ds(o, 16)] = sm[c]
        return tuple(sq)

    return lax.fori_loop(0, ngroups, gbody, ssq)


def _gr_body(pj0, pj1, gi0, gi1, pmax0, pmin0, gsum0, ssq0,
             pmax1, pmin1, gsum1, ssq1,
             idxb, rows, smax, smin, ssum, sem):
    wid = lax.axis_index("s") * 2 + lax.axis_index("c")

    pltpu.sync_copy(gi0.at[pl.ds(wid * 8, 8)], idxb)
    descs = [pltpu.async_copy(pj0.at[idxb.at[j]],
                              rows.at[pl.ds(j * 128, 128)], sem)
             for j in range(8)]
    for dsc in descs:
        dsc.wait()
    zero = jnp.zeros((16,), jnp.float32)
    sq = _gr_compute(rows, smax, smin, ssum, (zero,) * 4, 16, 64, 0)
    for c in range(4):
        smax[pl.ds(64 * 64 + c * 16, 16)] = sq[c]
    pltpu.sync_copy(smax.at[pl.ds(0, 64 * 64)], pmax0.at[pl.ds(wid * 64 * 64, 64 * 64)])
    pltpu.sync_copy(smin.at[pl.ds(0, 64 * 64)], pmin0.at[pl.ds(wid * 64 * 64, 64 * 64)])
    pltpu.sync_copy(ssum.at[pl.ds(0, 64 * 64)], gsum0.at[pl.ds(wid * 64 * 64, 64 * 64)])
    pltpu.sync_copy(smax.at[pl.ds(64 * 64, 64)], ssq0.at[pl.ds(wid * 64, 64)])

    sq = (zero,) * 4
    for h in range(2):
        pltpu.sync_copy(gi1.at[pl.ds(wid * 16 + h * 8, 8)], idxb)
        descs = [pltpu.async_copy(pj1.at[idxb.at[j]],
                                  rows.at[pl.ds(j * 128, 128)], sem)
                 for j in range(8)]
        for dsc in descs:
            dsc.wait()
        sq = _gr_compute(rows, smax, smin, ssum, sq, 32, 32, h * 32)
    for c in range(4):
        smax[pl.ds(64 * 64 + c * 16, 16)] = sq[c]
    pltpu.sync_copy(smax.at[pl.ds(0, 64 * 64)], pmax1.at[pl.ds(wid * 64 * 64, 64 * 64)])
    pltpu.sync_copy(smin.at[pl.ds(0, 64 * 64)], pmin1.at[pl.ds(wid * 64 * 64, 64 * 64)])
    pltpu.sync_copy(ssum.at[pl.ds(0, 64 * 64)], gsum1.at[pl.ds(wid * 64 * 64, 64 * 64)])
    pltpu.sync_copy(smax.at[pl.ds(64 * 64, 64)], ssq1.at[pl.ds(wid * 64, 64)])


def _run_gatherreduce(pj0, pj1, gi0, gi1):
    flat = jax.ShapeDtypeStruct((_B * _S * 64,), jnp.float32)
    sqs = jax.ShapeDtypeStruct((_NW * 64,), jnp.float32)
    f = functools.partial(
        pl.kernel, _gr_body,
        out_type=(flat, flat, flat, sqs, flat, flat, flat, sqs),
        mesh=_sc_mesh(),
        compiler_params=pltpu.CompilerParams(needs_layout_passes=False,
                                             use_tc_tiling_on_sc=False),
        scratch_types=[
            pltpu.VMEM((8, 128), jnp.int32),
            pltpu.VMEM((1024, 64), jnp.float32),
            pltpu.VMEM((64 * 64 + 64,), jnp.float32),
            pltpu.VMEM((64 * 64,), jnp.float32),
            pltpu.VMEM((64 * 64,), jnp.float32),
            pltpu.SemaphoreType.DMA,
        ],
    )
    return f()(pj0, pj1, gi0, gi1)


def _fin_body(newt_ref, pmax0_ref, pmin0_ref, gsum0_ref, ssq0_ref,
              pmax1_ref, pmin1_ref, gsum1_ref, ssq1_ref,
              w0x_ref, w1x_ref, g0_ref, b0_ref, g1_ref, b1_ref, out_ref):
    dn = (((0,), (0,)), ((), ()))
    for scale in range(2):
        K = _K[scale]
        pmax = (pmax0_ref, pmax1_ref)[scale]
        pmin = (pmin0_ref, pmin1_ref)[scale]
        gsum = (gsum0_ref, gsum1_ref)[scale]
        ssq = (ssq0_ref, ssq1_ref)[scale]
        wx = (w0x_ref, w1x_ref)[scale][...]
        gam = (g0_ref, g1_ref)[scale][...]
        bet = (b0_ref, b1_ref)[scale][...]
        R = _B * _S * K
        corr = [lax.dot_general(newt_ref[b], wx, dn,
                                preferred_element_type=jnp.float32)
                for b in range(_B)]
        sum_corr = sum(jnp.sum(c, axis=0) for c in corr)
        sum_gsum = jnp.sum(gsum[...], axis=(0, 1))
        cross = sum(jnp.sum(corr[b] * gsum[b], axis=0) for b in range(_B))
        sum_cc = sum(jnp.sum(c * c, axis=0) for c in corr)
        sumsq = jnp.sum(ssq[...], axis=0)
        mean = (sum_gsum - K * sum_corr) / R
        esq = (sumsq - 2.0 * cross + K * sum_cc) / R
        var = esq - mean * mean
        inv = lax.rsqrt(var + 1e-5)
        for b in range(_B):
            z = jnp.where(gam >= 0.0, pmax[b] - corr[b], pmin[b] - corr[b])
            y = jnp.maximum((z - mean) * inv * gam + bet, 0.0)
            out_ref[b, :, scale * 64:(scale + 1) * 64] = y


def _run_finalize(newt, pmax0, pmin0, gsum0, ssq0, pmax1, pmin1, gsum1, ssq1,
                  W0, W1, gamma0, beta0, gamma1, beta1):
    return pl.pallas_call(
        _fin_body,
        out_shape=jax.ShapeDtypeStruct((_B, _S, 128), jnp.float32),
    )(newt, pmax0, pmin0, gsum0, ssq0, pmax1, pmin1, gsum1, ssq1,
      W0[:3], W1[:3], gamma0, beta0, gamma1, beta1)


def kernel(xyz, features, W0, gamma0, beta0, W1, gamma1, beta1):
    xyzt = jnp.transpose(xyz, (0, 2, 1))
    xyzr = xyzt.reshape(_B, 3, _SUB, _LN)

    pj0, pj1 = _run_proj(xyz, features, W0, W1)
    new_xyz = _run_fps(xyzr)
    newt = jnp.transpose(new_xyz, (0, 2, 1))

    gi0, gi1 = _run_ballquery(xyzt.reshape(-1), newt.reshape(-1))
    gi0 = gi0.reshape(_B * _S * 16 // 128, 128)
    gi1 = gi1.reshape(_B * _S * 32 // 128, 128)

    (pmax0, pmin0, gsum0, ssq0, pmax1, pmin1, gsum1, ssq1) = _run_gatherreduce(
        pj0.reshape(_B * _N, 64), pj1.reshape(_B * _N, 64), gi0, gi1)

    shp = (_B, _S, 64)
    out = _run_finalize(newt, pmax0.reshape(shp), pmin0.reshape(shp),
                        gsum0.reshape(shp), ssq0.reshape(_NW, 64),
                        pmax1.reshape(shp), pmin1.reshape(shp),
                        gsum1.reshape(shp), ssq1.reshape(_NW, 64),
                        W0, W1, gamma0, beta0, gamma1, beta1)
    new_features = jnp.transpose(out, (0, 2, 1))
    return new_xyz, new_features


# speedup vs baseline: 55.3385x; 1.1241x over previous
import functools

import jax
import jax.numpy as jnp
from jax import lax
from jax.experimental import pallas as pl
from jax.experimental.pallas import tpu as pltpu
from jax.experimental.pallas import tpu_sc as plsc

_B = 2
_N = 16384
_S = 1024
_R2 = (0.2 * 0.2, 0.4 * 0.4)
_K = (16, 32)
_NW = 32
_SUB = 8
_LN = 2048


def _proj_body(xyz_ref, feat_ref, w0x_ref, w0f_ref, w1x_ref, w1f_ref,
               p0_ref, p1_ref):
    x = xyz_ref[0]
    f = feat_ref[0]
    p0_ref[0] = (jnp.dot(f, w0f_ref[...], preferred_element_type=jnp.float32)
                 + jnp.dot(x, w0x_ref[...], preferred_element_type=jnp.float32))
    p1_ref[0] = (jnp.dot(f, w1f_ref[...], preferred_element_type=jnp.float32)
                 + jnp.dot(x, w1x_ref[...], preferred_element_type=jnp.float32))


def _run_proj(xyz, features, W0, W1):
    w0x, w0f = W0[:3], W0[3:]
    w1x, w1f = W1[:3], W1[3:]
    nblk = _N // 1024
    grid = (_B, nblk)
    wspec3 = pl.BlockSpec((3, 64), lambda b, n: (0, 0))
    wspec128 = pl.BlockSpec((128, 64), lambda b, n: (0, 0))
    return pl.pallas_call(
        _proj_body,
        grid=grid,
        in_specs=[
            pl.BlockSpec((1, 1024, 3), lambda b, n: (b, n, 0)),
            pl.BlockSpec((1, 1024, 128), lambda b, n: (b, n, 0)),
            wspec3, wspec128, wspec3, wspec128,
        ],
        out_specs=[
            pl.BlockSpec((1, 1024, 64), lambda b, n: (b, n, 0)),
            pl.BlockSpec((1, 1024, 64), lambda b, n: (b, n, 0)),
        ],
        out_shape=[
            jax.ShapeDtypeStruct((_B, _N, 64), jnp.float32),
            jax.ShapeDtypeStruct((_B, _N, 64), jnp.float32),
        ],
    )(xyz, features, w0x, w0f, w1x, w1f)


def _fps_body(xyzr_ref, newxyz_ref, dist_ref):
    dist_ref[...] = jnp.full((_B, _SUB, _LN), 1e10, jnp.float32)

    def step(i, carry):
        nxt = []
        for b in range(_B):
            cx, cy, cz = carry[b]
            row = jnp.concatenate(
                [jnp.full((1, 1), cx), jnp.full((1, 1), cy),
                 jnp.full((1, 1), cz)], axis=1)
            newxyz_ref[b, pl.ds(i, 1), :] = row
            xb = xyzr_ref[b, 0]
            yb = xyzr_ref[b, 1]
            zb = xyzr_ref[b, 2]
            d = (xb - cx) ** 2 + (yb - cy) ** 2 + (zb - cz) ** 2
            dmin = jnp.minimum(dist_ref[b], d)
            dist_ref[b] = dmin
            m = jnp.max(dmin)
            sel = dmin == m
            nxt.append((jnp.max(jnp.where(sel, xb, -1e30)),
                        jnp.max(jnp.where(sel, yb, -1e30)),
                        jnp.max(jnp.where(sel, zb, -1e30))))
        return tuple(nxt)

    init = tuple((xyzr_ref[b, 0, 0, 0], xyzr_ref[b, 1, 0, 0],
                  xyzr_ref[b, 2, 0, 0]) for b in range(_B))
    lax.fori_loop(0, _S, step, init)


def _run_fps(xyzr):
    return pl.pallas_call(
        _fps_body,
        out_shape=jax.ShapeDtypeStruct((_B, _S, 3), jnp.float32),
        scratch_shapes=[pltpu.VMEM((_B, _SUB, _LN), jnp.float32)],
    )(xyzr)


def _sc_mesh():
    return plsc.VectorSubcoreMesh(core_axis_name="c", subcore_axis_name="s",
                                  num_cores=2, num_subcores=16)


def _bq_body(xyzt, newt, out0, out1, xs, ys, zs, cxr, cyr, czr, cbuf,
             ob0, ob1, sem):
    wid = lax.axis_index("s") * 2 + lax.axis_index("c")
    b = wid // 16
    part = wid % 16
    pltpu.sync_copy(xyzt.at[pl.ds((b * 3 + 0) * _N, _N)], xs)
    pltpu.sync_copy(xyzt.at[pl.ds((b * 3 + 1) * _N, _N)], ys)
    pltpu.sync_copy(xyzt.at[pl.ds((b * 3 + 2) * _N, _N)], zs)
    pltpu.sync_copy(newt.at[pl.ds((b * 3 + 0) * _S, _S)], cxr)
    pltpu.sync_copy(newt.at[pl.ds((b * 3 + 1) * _S, _S)], cyr)
    pltpu.sync_copy(newt.at[pl.ds((b * 3 + 2) * _S, _S)], czr)
    lane = lax.iota(jnp.int32, 16)
    base = b * _N

    for scale in range(2):
        r2 = jnp.float32(_R2[scale])
        K = _K[scale]
        ob = (ob0, ob1)[scale]

        def chunk_body(j, _, K=K, r2=r2, ob=ob):
            for t in range(4):
                s = j * 64 + part * 4 + t
                idxv = jnp.full((16,), 0, jnp.int32) + s
                cxv = plsc.load_gather(cxr, [idxv])
                cyv = plsc.load_gather(cyr, [idxv])
                czv = plsc.load_gather(czr, [idxv])
                cbuf[pl.ds(0, 16)] = jnp.full((16,), base, jnp.int32)

                def cond(carry):
                    jj, cnt = carry
                    return jnp.logical_and(cnt < K, jj < _N // 32)

                def body(carry, cxv=cxv, cyv=cyv, czv=czv):
                    jj, cnt = carry
                    o = jj * 32
                    xv0 = xs[pl.ds(o, 16)]
                    yv0 = ys[pl.ds(o, 16)]
                    zv0 = zs[pl.ds(o, 16)]
                    xv1 = xs[pl.ds(o + 16, 16)]
                    yv1 = ys[pl.ds(o + 16, 16)]
                    zv1 = zs[pl.ds(o + 16, 16)]
                    dx0 = xv0 - cxv
                    dy0 = yv0 - cyv
                    dz0 = zv0 - czv
                    d0 = dx0 * dx0 + dy0 * dy0 + dz0 * dz0
                    dx1 = xv1 - cxv
                    dy1 = yv1 - cyv
                    dz1 = zv1 - czv
                    d1 = dx1 * dx1 + dy1 * dy1 + dz1 * dz1
                    m0 = d0 < r2
                    m1 = d1 < r2
                    gi0 = lane + (o + base)
                    gi1 = lane + (o + 16 + base)
                    c0 = jnp.sum(m0.astype(jnp.int32))
                    c1 = jnp.sum(m1.astype(jnp.int32))
                    plsc.store_compressed(cbuf.at[pl.ds(cnt, 16)], gi0,
                                          mask=m0)
                    plsc.store_compressed(cbuf.at[pl.ds(cnt + c0, 16)], gi1,
                                          mask=m1)
                    return jj + 1, cnt + c0 + c1

                _, cnt = lax.while_loop(cond, body, (jnp.int32(0),
                                                     jnp.int32(0)))
                firstv = plsc.load_gather(cbuf, [jnp.zeros((16,), jnp.int32)])
                v0 = cbuf[pl.ds(0, 16)]
                slot = (j * 4 + t) * K
                if K == 16:
                    ob[pl.ds(slot, 16)] = jnp.where(lane < cnt, v0, firstv)
                else:
                    v1 = cbuf[pl.ds(16, 16)]
                    ob[pl.ds(slot, 16)] = jnp.where(lane < cnt, v0, firstv)
                    ob[pl.ds(slot + 16, 16)] = jnp.where(lane + 16 < cnt, v1,
                                                         firstv)
            return 0

        lax.fori_loop(0, 16, chunk_body, 0)
        out = (out0, out1)[scale]
        descs = [pltpu.async_copy(
            ob.at[pl.ds(j * 4 * K, 4 * K)],
            out.at[pl.ds(b * _S * K + (j * 64 + part * 4) * K, 4 * K)], sem)
            for j in range(16)]
        for dsc in descs:
            dsc.wait()


def _run_ballquery(xyzt, newt):
    f = functools.partial(
        pl.kernel, _bq_body,
        out_type=(jax.ShapeDtypeStruct((_B * _S * 16,), jnp.int32),
                  jax.ShapeDtypeStruct((_B * _S * 32,), jnp.int32)),
        mesh=_sc_mesh(),
        compiler_params=pltpu.CompilerParams(needs_layout_passes=False),
        scratch_types=[
            pltpu.VMEM((_N,), jnp.float32),
            pltpu.VMEM((_N,), jnp.float32),
            pltpu.VMEM((_N,), jnp.float32),
            pltpu.VMEM((_S,), jnp.float32),
            pltpu.VMEM((_S,), jnp.float32),
            pltpu.VMEM((_S,), jnp.float32),
            pltpu.VMEM((96,), jnp.int32),
            pltpu.VMEM((64 * 16,), jnp.int32),
            pltpu.VMEM((64 * 32,), jnp.int32),
            pltpu.SemaphoreType.DMA,
        ],
    )
    return f()(xyzt, newt)


def _gr_compute(rows, stage_max, stage_min, stage_sum, ssq, K, ngroups, goff):
    def gbody(g, ssq_c):
        first = g * K
        mx = [rows[first, pl.ds(c * 16, 16)] for c in range(4)]
        mn = list(mx)
        sm = list(mx)
        sq = [ssq_c[c] + mx[c] * mx[c] for c in range(4)]
        for r in range(1, K):
            for c in range(4):
                v = rows[first + r, pl.ds(c * 16, 16)]
                mx[c] = jnp.maximum(mx[c], v)
                mn[c] = jnp.minimum(mn[c], v)
                sm[c] = sm[c] + v
                sq[c] = sq[c] + v * v
        for c in range(4):
            o = (goff + g) * 64 + c * 16
            stage_max[pl.ds(o, 16)] = mx[c]
            stage_min[pl.ds(o, 16)] = mn[c]
            stage_sum[pl.ds(o, 16)] = sm[c]
        return tuple(sq)

    return lax.fori_loop(0, ngroups, gbody, ssq)


def _gr_body(pj0, pj1, gi0, gi1, pmax0, pmin0, gsum0, ssq0,
             pmax1, pmin1, gsum1, ssq1,
             idxb, rows, smax, smin, ssum, sem):
    wid = lax.axis_index("s") * 2 + lax.axis_index("c")

    pltpu.sync_copy(gi0.at[pl.ds(wid * 8, 8)], idxb)
    descs = [pltpu.async_copy(pj0.at[idxb.at[j]],
                              rows.at[pl.ds(j * 128, 128)], sem)
             for j in range(8)]
    for dsc in descs:
        dsc.wait()
    zero = jnp.zeros((16,), jnp.float32)
    sq = _gr_compute(rows, smax, smin, ssum, (zero,) * 4, 16, 64, 0)
    for c in range(4):
        smax[pl.ds(64 * 64 + c * 16, 16)] = sq[c]
    pltpu.sync_copy(smax.at[pl.ds(0, 64 * 64)], pmax0.at[pl.ds(wid * 64 * 64, 64 * 64)])
    pltpu.sync_copy(smin.at[pl.ds(0, 64 * 64)], pmin0.at[pl.ds(wid * 64 * 64, 64 * 64)])
    pltpu.sync_copy(ssum.at[pl.ds(0, 64 * 64)], gsum0.at[pl.ds(wid * 64 * 64, 64 * 64)])
    pltpu.sync_copy(smax.at[pl.ds(64 * 64, 64)], ssq0.at[pl.ds(wid * 64, 64)])

    sq = (zero,) * 4
    for h in range(2):
        pltpu.sync_copy(gi1.at[pl.ds(wid * 16 + h * 8, 8)], idxb)
        descs = [pltpu.async_copy(pj1.at[idxb.at[j]],
                                  rows.at[pl.ds(j * 128, 128)], sem)
                 for j in range(8)]
        for dsc in descs:
            dsc.wait()
        sq = _gr_compute(rows, smax, smin, ssum, sq, 32, 32, h * 32)
    for c in range(4):
        smax[pl.ds(64 * 64 + c * 16, 16)] = sq[c]
    pltpu.sync_copy(smax.at[pl.ds(0, 64 * 64)], pmax1.at[pl.ds(wid * 64 * 64, 64 * 64)])
    pltpu.sync_copy(smin.at[pl.ds(0, 64 * 64)], pmin1.at[pl.ds(wid * 64 * 64, 64 * 64)])
    pltpu.sync_copy(ssum.at[pl.ds(0, 64 * 64)], gsum1.at[pl.ds(wid * 64 * 64, 64 * 64)])
    pltpu.sync_copy(smax.at[pl.ds(64 * 64, 64)], ssq1.at[pl.ds(wid * 64, 64)])


def _run_gatherreduce(pj0, pj1, gi0, gi1):
    flat = jax.ShapeDtypeStruct((_B * _S * 64,), jnp.float32)
    sqs = jax.ShapeDtypeStruct((_NW * 64,), jnp.float32)
    f = functools.partial(
        pl.kernel, _gr_body,
        out_type=(flat, flat, flat, sqs, flat, flat, flat, sqs),
        mesh=_sc_mesh(),
        compiler_params=pltpu.CompilerParams(needs_layout_passes=False,
                                             use_tc_tiling_on_sc=False),
        scratch_types=[
            pltpu.VMEM((8, 128), jnp.int32),
            pltpu.VMEM((1024, 64), jnp.float32),
            pltpu.VMEM((64 * 64 + 64,), jnp.float32),
            pltpu.VMEM((64 * 64,), jnp.float32),
            pltpu.VMEM((64 * 64,), jnp.float32),
            pltpu.SemaphoreType.DMA,
        ],
    )
    return f()(pj0, pj1, gi0, gi1)


def _fin_body(newt_ref, pmax0_ref, pmin0_ref, gsum0_ref, ssq0_ref,
              pmax1_ref, pmin1_ref, gsum1_ref, ssq1_ref,
              w0x_ref, w1x_ref, g0_ref, b0_ref, g1_ref, b1_ref, out_ref):
    dn = (((0,), (0,)), ((), ()))
    for scale in range(2):
        K = _K[scale]
        pmax = (pmax0_ref, pmax1_ref)[scale]
        pmin = (pmin0_ref, pmin1_ref)[scale]
        gsum = (gsum0_ref, gsum1_ref)[scale]
        ssq = (ssq0_ref, ssq1_ref)[scale]
        wx = (w0x_ref, w1x_ref)[scale][...]
        gam = (g0_ref, g1_ref)[scale][...]
        bet = (b0_ref, b1_ref)[scale][...]
        R = _B * _S * K
        corr = [lax.dot_general(newt_ref[b], wx, dn,
                                preferred_element_type=jnp.float32)
                for b in range(_B)]
        sum_corr = sum(jnp.sum(c, axis=0) for c in corr)
        sum_gsum = jnp.sum(gsum[...], axis=(0, 1))
        cross = sum(jnp.sum(corr[b] * gsum[b], axis=0) for b in range(_B))
        sum_cc = sum(jnp.sum(c * c, axis=0) for c in corr)
        sumsq = jnp.sum(ssq[...], axis=0)
        mean = (sum_gsum - K * sum_corr) / R
        esq = (sumsq - 2.0 * cross + K * sum_cc) / R
        var = esq - mean * mean
        inv = lax.rsqrt(var + 1e-5)
        for b in range(_B):
            z = jnp.where(gam >= 0.0, pmax[b] - corr[b], pmin[b] - corr[b])
            y = jnp.maximum((z - mean) * inv * gam + bet, 0.0)
            out_ref[b, :, scale * 64:(scale + 1) * 64] = y


def _run_finalize(newt, pmax0, pmin0, gsum0, ssq0, pmax1, pmin1, gsum1, ssq1,
                  W0, W1, gamma0, beta0, gamma1, beta1):
    return pl.pallas_call(
        _fin_body,
        out_shape=jax.ShapeDtypeStruct((_B, _S, 128), jnp.float32),
    )(newt, pmax0, pmin0, gsum0, ssq0, pmax1, pmin1, gsum1, ssq1,
      W0[:3], W1[:3], gamma0, beta0, gamma1, beta1)


def kernel(xyz, features, W0, gamma0, beta0, W1, gamma1, beta1):
    xyzt = jnp.transpose(xyz, (0, 2, 1))
    xyzr = xyzt.reshape(_B, 3, _SUB, _LN)

    pj0, pj1 = _run_proj(xyz, features, W0, W1)
    new_xyz = _run_fps(xyzr)
    newt = jnp.transpose(new_xyz, (0, 2, 1))

    gi0, gi1 = _run_ballquery(xyzt.reshape(-1), newt.reshape(-1))
    gi0 = gi0.reshape(_B * _S * 16 // 128, 128)
    gi1 = gi1.reshape(_B * _S * 32 // 128, 128)

    (pmax0, pmin0, gsum0, ssq0, pmax1, pmin1, gsum1, ssq1) = _run_gatherreduce(
        pj0.reshape(_B * _N, 64), pj1.reshape(_B * _N, 64), gi0, gi1)

    shp = (_B, _S, 64)
    out = _run_finalize(newt, pmax0.reshape(shp), pmin0.reshape(shp),
                        gsum0.reshape(shp), ssq0.reshape(_NW, 64),
                        pmax1.reshape(shp), pmin1.reshape(shp),
                        gsum1.reshape(shp), ssq1.reshape(_NW, 64),
                        W0, W1, gamma0, beta0, gamma1, beta1)
    new_features = jnp.transpose(out, (0, 2, 1))
    return new_xyz, new_features


# speedup vs baseline: 68.2330x; 1.2330x over previous
import functools

import jax
import jax.numpy as jnp
from jax import lax
from jax.experimental import pallas as pl
from jax.experimental.pallas import tpu as pltpu
from jax.experimental.pallas import tpu_sc as plsc

_B = 2
_N = 16384
_S = 1024
_R2 = (0.2 * 0.2, 0.4 * 0.4)
_K = (16, 32)
_NW = 32
_SUB = 8
_LN = 2048


def _proj_body(xyz_ref, feat_ref, w0x_ref, w0f_ref, w1x_ref, w1f_ref,
               p0_ref, p1_ref):
    x = xyz_ref[0]
    f = feat_ref[0]
    p0_ref[0] = (jnp.dot(f, w0f_ref[...], preferred_element_type=jnp.float32)
                 + jnp.dot(x, w0x_ref[...], preferred_element_type=jnp.float32))
    p1_ref[0] = (jnp.dot(f, w1f_ref[...], preferred_element_type=jnp.float32)
                 + jnp.dot(x, w1x_ref[...], preferred_element_type=jnp.float32))


def _run_proj(xyz, features, W0, W1):
    w0x, w0f = W0[:3], W0[3:]
    w1x, w1f = W1[:3], W1[3:]
    nblk = _N // 1024
    grid = (_B, nblk)
    wspec3 = pl.BlockSpec((3, 64), lambda b, n: (0, 0))
    wspec128 = pl.BlockSpec((128, 64), lambda b, n: (0, 0))
    return pl.pallas_call(
        _proj_body,
        grid=grid,
        in_specs=[
            pl.BlockSpec((1, 1024, 3), lambda b, n: (b, n, 0)),
            pl.BlockSpec((1, 1024, 128), lambda b, n: (b, n, 0)),
            wspec3, wspec128, wspec3, wspec128,
        ],
        out_specs=[
            pl.BlockSpec((1, 1024, 64), lambda b, n: (b, n, 0)),
            pl.BlockSpec((1, 1024, 64), lambda b, n: (b, n, 0)),
        ],
        out_shape=[
            jax.ShapeDtypeStruct((_B, _N, 64), jnp.float32),
            jax.ShapeDtypeStruct((_B, _N, 64), jnp.float32),
        ],
    )(xyz, features, w0x, w0f, w1x, w1f)


def _fps_body(xyz0_ref, xyz1_ref, new0_ref, new1_ref, dist0_ref, dist1_ref):
    xyzs = (xyz0_ref, xyz1_ref)
    news = (new0_ref, new1_ref)
    dists = (dist0_ref, dist1_ref)
    for b in range(_B):
        dists[b][...] = jnp.full((_SUB, _LN), 1e10, jnp.float32)

    def argmax_tree(d, x, y, z):
        w = d.shape[1]
        while w > 128:
            h = w // 2
            c = d[:, :h] >= d[:, h:]
            d = jnp.where(c, d[:, :h], d[:, h:])
            x = jnp.where(c, x[:, :h], x[:, h:])
            y = jnp.where(c, y[:, :h], y[:, h:])
            z = jnp.where(c, z[:, :h], z[:, h:])
            w = h
        m = jnp.max(jnp.max(d, axis=0, keepdims=True), axis=1, keepdims=True)
        sel = d == m
        nx = jnp.where(sel, x, -1e30)
        ny = jnp.where(sel, y, -1e30)
        nz = jnp.where(sel, z, -1e30)
        red = lambda a: jnp.max(jnp.max(a, axis=0, keepdims=True), axis=1,
                                keepdims=True)
        return red(nx), red(ny), red(nz)

    def step(i, carry):
        nxt = []
        for b in range(_B):
            cx, cy, cz = carry[b]
            row = jnp.concatenate([cx, cy, cz], axis=1)
            news[b][pl.ds(i, 1), :] = row
            xb = xyzs[b][0]
            yb = xyzs[b][1]
            zb = xyzs[b][2]
            d = (xb - cx) ** 2 + (yb - cy) ** 2 + (zb - cz) ** 2
            dmin = jnp.minimum(dists[b][...], d)
            dists[b][...] = dmin
            nxt.append(argmax_tree(dmin, xb, yb, zb))
        return tuple(nxt)

    init = tuple((xyzs[b][0, 0:1, 0:1], xyzs[b][1, 0:1, 0:1],
                  xyzs[b][2, 0:1, 0:1]) for b in range(_B))
    lax.fori_loop(0, _S, step, init)


def _run_fps(xyzr):
    new0, new1 = pl.pallas_call(
        _fps_body,
        out_shape=[jax.ShapeDtypeStruct((_S, 3), jnp.float32),
                   jax.ShapeDtypeStruct((_S, 3), jnp.float32)],
        scratch_shapes=[pltpu.VMEM((_SUB, _LN), jnp.float32),
                        pltpu.VMEM((_SUB, _LN), jnp.float32)],
    )(xyzr[0], xyzr[1])
    return jnp.stack([new0, new1], axis=0)


def _sc_mesh():
    return plsc.VectorSubcoreMesh(core_axis_name="c", subcore_axis_name="s",
                                  num_cores=2, num_subcores=16)


def _bq_body(xyzt, newt, out0, out1, xs, ys, zs, cxr, cyr, czr, cbuf,
             ob0, ob1, sem):
    wid = lax.axis_index("s") * 2 + lax.axis_index("c")
    b = wid // 16
    part = wid % 16
    pltpu.sync_copy(xyzt.at[pl.ds((b * 3 + 0) * _N, _N)], xs)
    pltpu.sync_copy(xyzt.at[pl.ds((b * 3 + 1) * _N, _N)], ys)
    pltpu.sync_copy(xyzt.at[pl.ds((b * 3 + 2) * _N, _N)], zs)
    pltpu.sync_copy(newt.at[pl.ds((b * 3 + 0) * _S, _S)], cxr)
    pltpu.sync_copy(newt.at[pl.ds((b * 3 + 1) * _S, _S)], cyr)
    pltpu.sync_copy(newt.at[pl.ds((b * 3 + 2) * _S, _S)], czr)
    lane = lax.iota(jnp.int32, 16)
    base = b * _N

    for scale in range(2):
        r2 = jnp.float32(_R2[scale])
        K = _K[scale]
        ob = (ob0, ob1)[scale]

        def chunk_body(j, _, K=K, r2=r2, ob=ob):
            for t in range(4):
                s = j * 64 + part * 4 + t
                idxv = jnp.full((16,), 0, jnp.int32) + s
                cxv = plsc.load_gather(cxr, [idxv])
                cyv = plsc.load_gather(cyr, [idxv])
                czv = plsc.load_gather(czr, [idxv])
                cbuf[pl.ds(0, 16)] = jnp.full((16,), base, jnp.int32)

                def cond(carry):
                    jj, cnt = carry
                    return jnp.logical_and(cnt < K, jj < _N // 32)

                def body(carry, cxv=cxv, cyv=cyv, czv=czv):
                    jj, cnt = carry
                    o = jj * 32
                    xv0 = xs[pl.ds(o, 16)]
                    yv0 = ys[pl.ds(o, 16)]
                    zv0 = zs[pl.ds(o, 16)]
                    xv1 = xs[pl.ds(o + 16, 16)]
                    yv1 = ys[pl.ds(o + 16, 16)]
                    zv1 = zs[pl.ds(o + 16, 16)]
                    dx0 = xv0 - cxv
                    dy0 = yv0 - cyv
                    dz0 = zv0 - czv
                    d0 = dx0 * dx0 + dy0 * dy0 + dz0 * dz0
                    dx1 = xv1 - cxv
                    dy1 = yv1 - cyv
                    dz1 = zv1 - czv
                    d1 = dx1 * dx1 + dy1 * dy1 + dz1 * dz1
                    m0 = d0 < r2
                    m1 = d1 < r2
                    gi0 = lane + (o + base)
                    gi1 = lane + (o + 16 + base)
                    c0 = jnp.sum(m0.astype(jnp.int32))
                    c1 = jnp.sum(m1.astype(jnp.int32))
                    plsc.store_compressed(cbuf.at[pl.ds(cnt, 16)], gi0,
                                          mask=m0)
                    plsc.store_compressed(cbuf.at[pl.ds(cnt + c0, 16)], gi1,
                                          mask=m1)
                    return jj + 1, cnt + c0 + c1

                _, cnt = lax.while_loop(cond, body, (jnp.int32(0),
                                                     jnp.int32(0)))
                firstv = plsc.load_gather(cbuf, [jnp.zeros((16,), jnp.int32)])
                v0 = cbuf[pl.ds(0, 16)]
                slot = (j * 4 + t) * K
                if K == 16:
                    ob[pl.ds(slot, 16)] = jnp.where(lane < cnt, v0, firstv)
                else:
                    v1 = cbuf[pl.ds(16, 16)]
                    ob[pl.ds(slot, 16)] = jnp.where(lane < cnt, v0, firstv)
                    ob[pl.ds(slot + 16, 16)] = jnp.where(lane + 16 < cnt, v1,
                                                         firstv)
            return 0

        lax.fori_loop(0, 16, chunk_body, 0)
        out = (out0, out1)[scale]
        descs = [pltpu.async_copy(
            ob.at[pl.ds(j * 4 * K, 4 * K)],
            out.at[pl.ds(b * _S * K + (j * 64 + part * 4) * K, 4 * K)], sem)
            for j in range(16)]
        for dsc in descs:
            dsc.wait()


def _run_ballquery(xyzt, newt):
    f = functools.partial(
        pl.kernel, _bq_body,
        out_type=(jax.ShapeDtypeStruct((_B * _S * 16,), jnp.int32),
                  jax.ShapeDtypeStruct((_B * _S * 32,), jnp.int32)),
        mesh=_sc_mesh(),
        compiler_params=pltpu.CompilerParams(needs_layout_passes=False),
        scratch_types=[
            pltpu.VMEM((_N,), jnp.float32),
            pltpu.VMEM((_N,), jnp.float32),
            pltpu.VMEM((_N,), jnp.float32),
            pltpu.VMEM((_S,), jnp.float32),
            pltpu.VMEM((_S,), jnp.float32),
            pltpu.VMEM((_S,), jnp.float32),
            pltpu.VMEM((96,), jnp.int32),
            pltpu.VMEM((64 * 16,), jnp.int32),
            pltpu.VMEM((64 * 32,), jnp.int32),
            pltpu.SemaphoreType.DMA,
        ],
    )
    return f()(xyzt, newt)


def _gr_compute(rows, stage_max, stage_min, stage_sum, ssq, K, ngroups, goff):
    def gbody(g, ssq_c):
        first = g * K
        mx = [rows[first, pl.ds(c * 16, 16)] for c in range(4)]
        mn = list(mx)
        sm = list(mx)
        sq = [ssq_c[c] + mx[c] * mx[c] for c in range(4)]
        for r in range(1, K):
            for c in range(4):
                v = rows[first + r, pl.ds(c * 16, 16)]
                mx[c] = jnp.maximum(mx[c], v)
                mn[c] = jnp.minimum(mn[c], v)
                sm[c] = sm[c] + v
                sq[c] = sq[c] + v * v
        for c in range(4):
            o = (goff + g) * 64 + c * 16
            stage_max[pl.ds(o, 16)] = mx[c]
            stage_min[pl.ds(o, 16)] = mn[c]
            stage_sum[pl.ds(o, 16)] = sm[c]
        return tuple(sq)

    return lax.fori_loop(0, ngroups, gbody, ssq)


def _gr_body(pj0, pj1, gi0, gi1, pmax0, pmin0, gsum0, ssq0,
             pmax1, pmin1, gsum1, ssq1,
             idxb, rows, smax, smin, ssum, sem):
    wid = lax.axis_index("s") * 2 + lax.axis_index("c")

    pltpu.sync_copy(gi0.at[pl.ds(wid * 8, 8)], idxb)
    descs = [pltpu.async_copy(pj0.at[idxb.at[j]],
                              rows.at[pl.ds(j * 128, 128)], sem)
             for j in range(8)]
    for dsc in descs:
        dsc.wait()
    zero = jnp.zeros((16,), jnp.float32)
    sq = _gr_compute(rows, smax, smin, ssum, (zero,) * 4, 16, 64, 0)
    for c in range(4):
        smax[pl.ds(64 * 64 + c * 16, 16)] = sq[c]
    pltpu.sync_copy(smax.at[pl.ds(0, 64 * 64)], pmax0.at[pl.ds(wid * 64 * 64, 64 * 64)])
    pltpu.sync_copy(smin.at[pl.ds(0, 64 * 64)], pmin0.at[pl.ds(wid * 64 * 64, 64 * 64)])
    pltpu.sync_copy(ssum.at[pl.ds(0, 64 * 64)], gsum0.at[pl.ds(wid * 64 * 64, 64 * 64)])
    pltpu.sync_copy(smax.at[pl.ds(64 * 64, 64)], ssq0.at[pl.ds(wid * 64, 64)])

    sq = (zero,) * 4
    for h in range(2):
        pltpu.sync_copy(gi1.at[pl.ds(wid * 16 + h * 8, 8)], idxb)
        descs = [pltpu.async_copy(pj1.at[idxb.at[j]],
                                  rows.at[pl.ds(j * 128, 128)], sem)
                 for j in range(8)]
        for dsc in descs:
            dsc.wait()
        sq = _gr_compute(rows, smax, smin, ssum, sq, 32, 32, h * 32)
    for c in range(4):
        smax[pl.ds(64 * 64 + c * 16, 16)] = sq[c]
    pltpu.sync_copy(smax.at[pl.ds(0, 64 * 64)], pmax1.at[pl.ds(wid * 64 * 64, 64 * 64)])
    pltpu.sync_copy(smin.at[pl.ds(0, 64 * 64)], pmin1.at[pl.ds(wid * 64 * 64, 64 * 64)])
    pltpu.sync_copy(ssum.at[pl.ds(0, 64 * 64)], gsum1.at[pl.ds(wid * 64 * 64, 64 * 64)])
    pltpu.sync_copy(smax.at[pl.ds(64 * 64, 64)], ssq1.at[pl.ds(wid * 64, 64)])


def _run_gatherreduce(pj0, pj1, gi0, gi1):
    flat = jax.ShapeDtypeStruct((_B * _S * 64,), jnp.float32)
    sqs = jax.ShapeDtypeStruct((_NW * 64,), jnp.float32)
    f = functools.partial(
        pl.kernel, _gr_body,
        out_type=(flat, flat, flat, sqs, flat, flat, flat, sqs),
        mesh=_sc_mesh(),
        compiler_params=pltpu.CompilerParams(needs_layout_passes=False,
                                             use_tc_tiling_on_sc=False),
        scratch_types=[
            pltpu.VMEM((8, 128), jnp.int32),
            pltpu.VMEM((1024, 64), jnp.float32),
            pltpu.VMEM((64 * 64 + 64,), jnp.float32),
            pltpu.VMEM((64 * 64,), jnp.float32),
            pltpu.VMEM((64 * 64,), jnp.float32),
            pltpu.SemaphoreType.DMA,
        ],
    )
    return f()(pj0, pj1, gi0, gi1)


def _fin_body(newt_ref, pmax0_ref, pmin0_ref, gsum0_ref, ssq0_ref,
              pmax1_ref, pmin1_ref, gsum1_ref, ssq1_ref,
              w0x_ref, w1x_ref, g0_ref, b0_ref, g1_ref, b1_ref, out_ref):
    dn = (((0,), (0,)), ((), ()))
    for scale in range(2):
        K = _K[scale]
        pmax = (pmax0_ref, pmax1_ref)[scale]
        pmin = (pmin0_ref, pmin1_ref)[scale]
        gsum = (gsum0_ref, gsum1_ref)[scale]
        ssq = (ssq0_ref, ssq1_ref)[scale]
        wx = (w0x_ref, w1x_ref)[scale][...]
        gam = (g0_ref, g1_ref)[scale][...]
        bet = (b0_ref, b1_ref)[scale][...]
        R = _B * _S * K
        corr = [lax.dot_general(newt_ref[b], wx, dn,
                                preferred_element_type=jnp.float32)
                for b in range(_B)]
        sum_corr = sum(jnp.sum(c, axis=0) for c in corr)
        sum_gsum = jnp.sum(gsum[...], axis=(0, 1))
        cross = sum(jnp.sum(corr[b] * gsum[b], axis=0) for b in range(_B))
        sum_cc = sum(jnp.sum(c * c, axis=0) for c in corr)
        sumsq = jnp.sum(ssq[...], axis=0)
        mean = (sum_gsum - K * sum_corr) / R
        esq = (sumsq - 2.0 * cross + K * sum_cc) / R
        var = esq - mean * mean
        inv = lax.rsqrt(var + 1e-5)
        for b in range(_B):
            z = jnp.where(gam >= 0.0, pmax[b] - corr[b], pmin[b] - corr[b])
            y = jnp.maximum((z - mean) * inv * gam + bet, 0.0)
            out_ref[b, :, scale * 64:(scale + 1) * 64] = y


def _run_finalize(newt, pmax0, pmin0, gsum0, ssq0, pmax1, pmin1, gsum1, ssq1,
                  W0, W1, gamma0, beta0, gamma1, beta1):
    return pl.pallas_call(
        _fin_body,
        out_shape=jax.ShapeDtypeStruct((_B, _S, 128), jnp.float32),
    )(newt, pmax0, pmin0, gsum0, ssq0, pmax1, pmin1, gsum1, ssq1,
      W0[:3], W1[:3], gamma0, beta0, gamma1, beta1)


def kernel(xyz, features, W0, gamma0, beta0, W1, gamma1, beta1):
    xyzt = jnp.transpose(xyz, (0, 2, 1))
    xyzr = xyzt.reshape(_B, 3, _SUB, _LN)

    pj0, pj1 = _run_proj(xyz, features, W0, W1)
    new_xyz = _run_fps(xyzr)
    newt = jnp.transpose(new_xyz, (0, 2, 1))

    gi0, gi1 = _run_ballquery(xyzt.reshape(-1), newt.reshape(-1))
    gi0 = gi0.reshape(_B * _S * 16 // 128, 128)
    gi1 = gi1.reshape(_B * _S * 32 // 128, 128)

    (pmax0, pmin0, gsum0, ssq0, pmax1, pmin1, gsum1, ssq1) = _run_gatherreduce(
        pj0.reshape(_B * _N, 64), pj1.reshape(_B * _N, 64), gi0, gi1)

    shp = (_B, _S, 64)
    out = _run_finalize(newt, pmax0.reshape(shp), pmin0.reshape(shp),
                        gsum0.reshape(shp), ssq0.reshape(_NW, 64),
                        pmax1.reshape(shp), pmin1.reshape(shp),
                        gsum1.reshape(shp), ssq1.reshape(_NW, 64),
                        W0, W1, gamma0, beta0, gamma1, beta1)
    new_features = jnp.transpose(out, (0, 2, 1))
    return new_xyz, new_features


# speedup vs baseline: 72.5087x; 1.0627x over previous
import functools

import jax
import jax.numpy as jnp
from jax import lax
from jax.experimental import pallas as pl
from jax.experimental.pallas import tpu as pltpu
from jax.experimental.pallas import tpu_sc as plsc

_B = 2
_N = 16384
_S = 1024
_R2 = (0.2 * 0.2, 0.4 * 0.4)
_K = (16, 32)
_NW = 32
_SUB = 8
_LN = 2048


def _proj_body(xyz_ref, feat_ref, w0x_ref, w0f_ref, w1x_ref, w1f_ref,
               p0_ref, p1_ref):
    x = xyz_ref[0]
    f = feat_ref[0]
    p0_ref[0] = (jnp.dot(f, w0f_ref[...], preferred_element_type=jnp.float32)
                 + jnp.dot(x, w0x_ref[...], preferred_element_type=jnp.float32))
    p1_ref[0] = (jnp.dot(f, w1f_ref[...], preferred_element_type=jnp.float32)
                 + jnp.dot(x, w1x_ref[...], preferred_element_type=jnp.float32))


def _run_proj(xyz, features, W0, W1):
    w0x, w0f = W0[:3], W0[3:]
    w1x, w1f = W1[:3], W1[3:]
    nblk = _N // 1024
    grid = (_B, nblk)
    wspec3 = pl.BlockSpec((3, 64), lambda b, n: (0, 0))
    wspec128 = pl.BlockSpec((128, 64), lambda b, n: (0, 0))
    return pl.pallas_call(
        _proj_body,
        grid=grid,
        in_specs=[
            pl.BlockSpec((1, 1024, 3), lambda b, n: (b, n, 0)),
            pl.BlockSpec((1, 1024, 128), lambda b, n: (b, n, 0)),
            wspec3, wspec128, wspec3, wspec128,
        ],
        out_specs=[
            pl.BlockSpec((1, 1024, 64), lambda b, n: (b, n, 0)),
            pl.BlockSpec((1, 1024, 64), lambda b, n: (b, n, 0)),
        ],
        out_shape=[
            jax.ShapeDtypeStruct((_B, _N, 64), jnp.float32),
            jax.ShapeDtypeStruct((_B, _N, 64), jnp.float32),
        ],
    )(xyz, features, w0x, w0f, w1x, w1f)


def _fps_body(xyz0_ref, xyz1_ref, new0_ref, new1_ref, dist0_ref, dist1_ref):
    xyzs = (xyz0_ref, xyz1_ref)
    news = (new0_ref, new1_ref)
    dists = (dist0_ref, dist1_ref)
    for b in range(_B):
        dists[b][...] = jnp.full((_SUB, _LN), 1e10, jnp.float32)

    def argmax_tree(d, x, y, z):
        w = d.shape[1]
        while w > 128:
            h = w // 2
            c = d[:, :h] >= d[:, h:]
            d = jnp.where(c, d[:, :h], d[:, h:])
            x = jnp.where(c, x[:, :h], x[:, h:])
            y = jnp.where(c, y[:, :h], y[:, h:])
            z = jnp.where(c, z[:, :h], z[:, h:])
            w = h
        m = jnp.max(jnp.max(d, axis=0, keepdims=True), axis=1, keepdims=True)
        sel = d == m
        nx = jnp.where(sel, x, -1e30)
        ny = jnp.where(sel, y, -1e30)
        nz = jnp.where(sel, z, -1e30)
        red = lambda a: jnp.max(jnp.max(a, axis=0, keepdims=True), axis=1,
                                keepdims=True)[0, 0]
        return red(nx), red(ny), red(nz)

    def step(i, carry):
        nxt = []
        for b in range(_B):
            cx, cy, cz = carry[b]
            row = jnp.concatenate(
                [jnp.full((1, 1), cx), jnp.full((1, 1), cy),
                 jnp.full((1, 1), cz)], axis=1)
            news[b][pl.ds(i, 1), :] = row
            xb = xyzs[b][0]
            yb = xyzs[b][1]
            zb = xyzs[b][2]
            d = (xb - cx) ** 2 + (yb - cy) ** 2 + (zb - cz) ** 2
            dmin = jnp.minimum(dists[b][...], d)
            dists[b][...] = dmin
            nxt.append(argmax_tree(dmin, xb, yb, zb))
        return tuple(nxt)

    init = tuple((xyzs[b][0, 0, 0], xyzs[b][1, 0, 0],
                  xyzs[b][2, 0, 0]) for b in range(_B))
    lax.fori_loop(0, _S, step, init)


def _run_fps(xyzr):
    new0, new1 = pl.pallas_call(
        _fps_body,
        out_shape=[jax.ShapeDtypeStruct((_S, 3), jnp.float32),
                   jax.ShapeDtypeStruct((_S, 3), jnp.float32)],
        scratch_shapes=[pltpu.VMEM((_SUB, _LN), jnp.float32),
                        pltpu.VMEM((_SUB, _LN), jnp.float32)],
    )(xyzr[0], xyzr[1])
    return jnp.stack([new0, new1], axis=0)


def _sc_mesh():
    return plsc.VectorSubcoreMesh(core_axis_name="c", subcore_axis_name="s",
                                  num_cores=2, num_subcores=16)


def _bq_body(xyzt, newt, out0, out1, xs, ys, zs, cxr, cyr, czr, cbuf,
             ob0, ob1, sem):
    wid = lax.axis_index("s") * 2 + lax.axis_index("c")
    b = wid // 16
    part = wid % 16
    pltpu.sync_copy(xyzt.at[pl.ds((b * 3 + 0) * _N, _N)], xs)
    pltpu.sync_copy(xyzt.at[pl.ds((b * 3 + 1) * _N, _N)], ys)
    pltpu.sync_copy(xyzt.at[pl.ds((b * 3 + 2) * _N, _N)], zs)
    pltpu.sync_copy(newt.at[pl.ds((b * 3 + 0) * _S, _S)], cxr)
    pltpu.sync_copy(newt.at[pl.ds((b * 3 + 1) * _S, _S)], cyr)
    pltpu.sync_copy(newt.at[pl.ds((b * 3 + 2) * _S, _S)], czr)
    lane = lax.iota(jnp.int32, 16)
    base = b * _N

    for scale in range(2):
        r2 = jnp.float32(_R2[scale])
        K = _K[scale]
        ob = (ob0, ob1)[scale]

        def chunk_body(j, _, K=K, r2=r2, ob=ob):
            for t in range(4):
                s = j * 64 + part * 4 + t
                idxv = jnp.full((16,), 0, jnp.int32) + s
                cxv = plsc.load_gather(cxr, [idxv])
                cyv = plsc.load_gather(cyr, [idxv])
                czv = plsc.load_gather(czr, [idxv])
                cbuf[pl.ds(0, 16)] = jnp.full((16,), base, jnp.int32)

                def cond(carry):
                    jj, cnt = carry
                    return jnp.logical_and(cnt < K, jj < _N // 64)

                def body(carry, cxv=cxv, cyv=cyv, czv=czv):
                    jj, cnt = carry
                    o = jj * 64
                    ms, cs = [], []
                    for u in range(4):
                        xv = xs[pl.ds(o + 16 * u, 16)]
                        yv = ys[pl.ds(o + 16 * u, 16)]
                        zv = zs[pl.ds(o + 16 * u, 16)]
                        dx = xv - cxv
                        dy = yv - cyv
                        dz = zv - czv
                        d = dx * dx + dy * dy + dz * dz
                        m = d < r2
                        ms.append(m)
                        cs.append(jnp.sum(m.astype(jnp.int32)))
                    off = cnt
                    for u in range(4):
                        gi = lane + (o + 16 * u + base)
                        plsc.store_compressed(cbuf.at[pl.ds(off, 16)], gi,
                                              mask=ms[u])
                        off = off + cs[u]
                    return jj + 1, off

                _, cnt = lax.while_loop(cond, body, (jnp.int32(0),
                                                     jnp.int32(0)))
                firstv = plsc.load_gather(cbuf, [jnp.zeros((16,), jnp.int32)])
                v0 = cbuf[pl.ds(0, 16)]
                slot = (j * 4 + t) * K
                if K == 16:
                    ob[pl.ds(slot, 16)] = jnp.where(lane < cnt, v0, firstv)
                else:
                    v1 = cbuf[pl.ds(16, 16)]
                    ob[pl.ds(slot, 16)] = jnp.where(lane < cnt, v0, firstv)
                    ob[pl.ds(slot + 16, 16)] = jnp.where(lane + 16 < cnt, v1,
                                                         firstv)
            return 0

        lax.fori_loop(0, 16, chunk_body, 0)
        out = (out0, out1)[scale]
        descs = [pltpu.async_copy(
            ob.at[pl.ds(j * 4 * K, 4 * K)],
            out.at[pl.ds(b * _S * K + (j * 64 + part * 4) * K, 4 * K)], sem)
            for j in range(16)]
        for dsc in descs:
            dsc.wait()


def _run_ballquery(xyzt, newt):
    f = functools.partial(
        pl.kernel, _bq_body,
        out_type=(jax.ShapeDtypeStruct((_B * _S * 16,), jnp.int32),
                  jax.ShapeDtypeStruct((_B * _S * 32,), jnp.int32)),
        mesh=_sc_mesh(),
        compiler_params=pltpu.CompilerParams(needs_layout_passes=False),
        scratch_types=[
            pltpu.VMEM((_N,), jnp.float32),
            pltpu.VMEM((_N,), jnp.float32),
            pltpu.VMEM((_N,), jnp.float32),
            pltpu.VMEM((_S,), jnp.float32),
            pltpu.VMEM((_S,), jnp.float32),
            pltpu.VMEM((_S,), jnp.float32),
            pltpu.VMEM((128,), jnp.int32),
            pltpu.VMEM((64 * 16,), jnp.int32),
            pltpu.VMEM((64 * 32,), jnp.int32),
            pltpu.SemaphoreType.DMA,
        ],
    )
    return f()(xyzt, newt)


def _gr_compute(rows, stage_max, stage_min, stage_sum, ssq, K, ngroups, goff):
    def gbody(g, ssq_c):
        first = g * K
        mx = [rows[first, pl.ds(c * 16, 16)] for c in range(4)]
        mn = list(mx)
        sm = list(mx)
        sq = [ssq_c[c] + mx[c] * mx[c] for c in range(4)]
        for r in range(1, K):
            for c in range(4):
                v = rows[first + r, pl.ds(c * 16, 16)]
                mx[c] = jnp.maximum(mx[c], v)
                mn[c] = jnp.minimum(mn[c], v)
                sm[c] = sm[c] + v
                sq[c] = sq[c] + v * v
        for c in range(4):
            o = (goff + g) * 64 + c * 16
            stage_max[pl.ds(o, 16)] = mx[c]
            stage_min[pl.ds(o, 16)] = mn[c]
            stage_sum[pl.ds(o, 16)] = sm[c]
        return tuple(sq)

    return lax.fori_loop(0, ngroups, gbody, ssq)


def _gr_body(pj0, pj1, gi0, gi1, pmax0, pmin0, gsum0, ssq0,
             pmax1, pmin1, gsum1, ssq1,
             idxb, rows, smax, smin, ssum, sem):
    wid = lax.axis_index("s") * 2 + lax.axis_index("c")

    pltpu.sync_copy(gi0.at[pl.ds(wid * 8, 8)], idxb)
    descs = [pltpu.async_copy(pj0.at[idxb.at[j]],
                              rows.at[pl.ds(j * 128, 128)], sem)
             for j in range(8)]
    for dsc in descs:
        dsc.wait()
    zero = jnp.zeros((16,), jnp.float32)
    sq = _gr_compute(rows, smax, smin, ssum, (zero,) * 4, 16, 64, 0)
    for c in range(4):
        smax[pl.ds(64 * 64 + c * 16, 16)] = sq[c]
    pltpu.sync_copy(smax.at[pl.ds(0, 64 * 64)], pmax0.at[pl.ds(wid * 64 * 64, 64 * 64)])
    pltpu.sync_copy(smin.at[pl.ds(0, 64 * 64)], pmin0.at[pl.ds(wid * 64 * 64, 64 * 64)])
    pltpu.sync_copy(ssum.at[pl.ds(0, 64 * 64)], gsum0.at[pl.ds(wid * 64 * 64, 64 * 64)])
    pltpu.sync_copy(smax.at[pl.ds(64 * 64, 64)], ssq0.at[pl.ds(wid * 64, 64)])

    sq = (zero,) * 4
    for h in range(2):
        pltpu.sync_copy(gi1.at[pl.ds(wid * 16 + h * 8, 8)], idxb)
        descs = [pltpu.async_copy(pj1.at[idxb.at[j]],
                                  rows.at[pl.ds(j * 128, 128)], sem)
                 for j in range(8)]
        for dsc in descs:
            dsc.wait()
        sq = _gr_compute(rows, smax, smin, ssum, sq, 32, 32, h * 32)
    for c in range(4):
        smax[pl.ds(64 * 64 + c * 16, 16)] = sq[c]
    pltpu.sync_copy(smax.at[pl.ds(0, 64 * 64)], pmax1.at[pl.ds(wid * 64 * 64, 64 * 64)])
    pltpu.sync_copy(smin.at[pl.ds(0, 64 * 64)], pmin1.at[pl.ds(wid * 64 * 64, 64 * 64)])
    pltpu.sync_copy(ssum.at[pl.ds(0, 64 * 64)], gsum1.at[pl.ds(wid * 64 * 64, 64 * 64)])
    pltpu.sync_copy(smax.at[pl.ds(64 * 64, 64)], ssq1.at[pl.ds(wid * 64, 64)])


def _run_gatherreduce(pj0, pj1, gi0, gi1):
    flat = jax.ShapeDtypeStruct((_B * _S * 64,), jnp.float32)
    sqs = jax.ShapeDtypeStruct((_NW * 64,), jnp.float32)
    f = functools.partial(
        pl.kernel, _gr_body,
        out_type=(flat, flat, flat, sqs, flat, flat, flat, sqs),
        mesh=_sc_mesh(),
        compiler_params=pltpu.CompilerParams(needs_layout_passes=False,
                                             use_tc_tiling_on_sc=False),
        scratch_types=[
            pltpu.VMEM((8, 128), jnp.int32),
            pltpu.VMEM((1024, 64), jnp.float32),
            pltpu.VMEM((64 * 64 + 64,), jnp.float32),
            pltpu.VMEM((64 * 64,), jnp.float32),
            pltpu.VMEM((64 * 64,), jnp.float32),
            pltpu.SemaphoreType.DMA,
        ],
    )
    return f()(pj0, pj1, gi0, gi1)


def _fin_body(newt_ref, pmax0_ref, pmin0_ref, gsum0_ref, ssq0_ref,
              pmax1_ref, pmin1_ref, gsum1_ref, ssq1_ref,
              w0x_ref, w1x_ref, g0_ref, b0_ref, g1_ref, b1_ref, out_ref):
    dn = (((0,), (0,)), ((), ()))
    for scale in range(2):
        K = _K[scale]
        pmax = (pmax0_ref, pmax1_ref)[scale]
        pmin = (pmin0_ref, pmin1_ref)[scale]
        gsum = (gsum0_ref, gsum1_ref)[scale]
        ssq = (ssq0_ref, ssq1_ref)[scale]
        wx = (w0x_ref, w1x_ref)[scale][...]
        gam = (g0_ref, g1_ref)[scale][...]
        bet = (b0_ref, b1_ref)[scale][...]
        R = _B * _S * K
        corr = [lax.dot_general(newt_ref[b], wx, dn,
                                preferred_element_type=jnp.float32)
                for b in range(_B)]
        sum_corr = sum(jnp.sum(c, axis=0) for c in corr)
        sum_gsum = jnp.sum(gsum[...], axis=(0, 1))
        cross = sum(jnp.sum(corr[b] * gsum[b], axis=0) for b in range(_B))
        sum_cc = sum(jnp.sum(c * c, axis=0) for c in corr)
        sumsq = jnp.sum(ssq[...], axis=0)
        mean = (sum_gsum - K * sum_corr) / R
        esq = (sumsq - 2.0 * cross + K * sum_cc) / R
        var = esq - mean * mean
        inv = lax.rsqrt(var + 1e-5)
        for b in range(_B):
            z = jnp.where(gam >= 0.0, pmax[b] - corr[b], pmin[b] - corr[b])
            y = jnp.maximum((z - mean) * inv * gam + bet, 0.0)
            out_ref[b, :, scale * 64:(scale + 1) * 64] = y


def _run_finalize(newt, pmax0, pmin0, gsum0, ssq0, pmax1, pmin1, gsum1, ssq1,
                  W0, W1, gamma0, beta0, gamma1, beta1):
    return pl.pallas_call(
        _fin_body,
        out_shape=jax.ShapeDtypeStruct((_B, _S, 128), jnp.float32),
    )(newt, pmax0, pmin0, gsum0, ssq0, pmax1, pmin1, gsum1, ssq1,
      W0[:3], W1[:3], gamma0, beta0, gamma1, beta1)


def kernel(xyz, features, W0, gamma0, beta0, W1, gamma1, beta1):
    xyzt = jnp.transpose(xyz, (0, 2, 1))
    xyzr = xyzt.reshape(_B, 3, _SUB, _LN)

    pj0, pj1 = _run_proj(xyz, features, W0, W1)
    new_xyz = _run_fps(xyzr)
    newt = jnp.transpose(new_xyz, (0, 2, 1))

    gi0, gi1 = _run_ballquery(xyzt.reshape(-1), newt.reshape(-1))
    gi0 = gi0.reshape(_B * _S * 16 // 128, 128)
    gi1 = gi1.reshape(_B * _S * 32 // 128, 128)

    (pmax0, pmin0, gsum0, ssq0, pmax1, pmin1, gsum1, ssq1) = _run_gatherreduce(
        pj0.reshape(_B * _N, 64), pj1.reshape(_B * _N, 64), gi0, gi1)

    shp = (_B, _S, 64)
    out = _run_finalize(newt, pmax0.reshape(shp), pmin0.reshape(shp),
                        gsum0.reshape(shp), ssq0.reshape(_NW, 64),
                        pmax1.reshape(shp), pmin1.reshape(shp),
                        gsum1.reshape(shp), ssq1.reshape(_NW, 64),
                        W0, W1, gamma0, beta0, gamma1, beta1)
    new_features = jnp.transpose(out, (0, 2, 1))
    return new_xyz, new_features
